# Initial kernel scaffold; baseline (speedup 1.0000x reference)
import jax
import jax.numpy as jnp
from jax.experimental import pallas as pl


def kernel(fnode, fmess, node_graph, mess_graph, scope, emb, Wz_w, Wz_b, Wr_w, Ur_w, Ur_b, Wh_w, Wh_b, out_w, out_b):
    raise NotImplementedError("write your pallas kernel here")


# speedup vs baseline: 1.0335x; 1.0335x over previous
import functools

import jax
import jax.numpy as jnp
from jax import lax
from jax.experimental import pallas as pl
from jax.experimental.pallas import tpu as pltpu
from jax.experimental.pallas import tpu_sc as plsc

N = 10000
E = 20000
K = 8
H = 128
DEPTH = 10
B = 256

NC, NS = 2, 16
NW = NC * NS
EP = 20480
EPW = EP // NW
CE = 16
CI = CE * K
NCH = EPW // CE
HV = H // 16

_mesh = plsc.VectorSubcoreMesh(
    core_axis_name="c", subcore_axis_name="s", num_cores=NC, num_subcores=NS)
_sc_params = pltpu.CompilerParams(needs_layout_passes=False)


def _wid():
    return lax.axis_index("s") * NC + lax.axis_index("c")


def _sigmoid16(t):
    return 1.0 / (1.0 + jnp.exp(-t))


@functools.partial(
    pl.kernel,
    out_type=jax.ShapeDtypeStruct((EP, H), jnp.float32),
    mesh=_mesh,
    compiler_params=_sc_params,
    scratch_types=[
        pltpu.VMEM((N,), jnp.int32),
        pltpu.VMEM((EPW,), jnp.int32),
        pltpu.VMEM((CE,), jnp.int32),
        pltpu.VMEM((CE, H), jnp.float32),
        pltpu.SemaphoreType.DMA,
    ],
)
def _sc_embed(fnode_hbm, fmess_hbm, emb_hbm, x_hbm,
              fnode_v, fmess_v, idx_v, rows_v, sem):
    base = _wid() * EPW
    pltpu.sync_copy(fnode_hbm, fnode_v)
    pltpu.sync_copy(fmess_hbm.at[pl.ds(base, EPW)], fmess_v)

    def chunk(c, carry):
        fm = fmess_v[pl.ds(c * CE, CE)]
        idx_v[...] = plsc.load_gather(fnode_v, [fm])
        pltpu.async_copy(emb_hbm.at[idx_v], rows_v, sem).wait()
        pltpu.sync_copy(rows_v, x_hbm.at[pl.ds(base + c * CE, CE)])
        return carry

    lax.fori_loop(0, NCH, chunk, 0)


@functools.partial(
    pl.kernel,
    out_type=(jax.ShapeDtypeStruct((EP, H), jnp.float32),
              jax.ShapeDtypeStruct((EP, H), jnp.float32)),
    mesh=_mesh,
    compiler_params=_sc_params,
    scratch_types=[
        pltpu.VMEM((CI,), jnp.int32),
        pltpu.VMEM((CI, H), jnp.float32),
        pltpu.VMEM((CI, H), jnp.float32),
        pltpu.VMEM((CE, H), jnp.float32),
        pltpu.VMEM((CE, H), jnp.float32),
        pltpu.VMEM((CE, H), jnp.float32),
        pltpu.SemaphoreType.DMA,
    ],
)
def _sc_gather(mg_hbm, h_hbm, hu_hbm, r1_hbm, sh_hbm, sg_hbm,
               idx_v, hrows_v, urows_v, r1_v, os_v, og_v, sem):
    base = _wid() * EPW

    def chunk(c, carry):
        row0 = base + c * CE
        pltpu.sync_copy(mg_hbm.at[pl.ds(row0 * K, CI)], idx_v)
        cp_h = pltpu.async_copy(h_hbm.at[idx_v], hrows_v, sem)
        cp_u = pltpu.async_copy(hu_hbm.at[idx_v], urows_v, sem)
        cp_r = pltpu.async_copy(r1_hbm.at[pl.ds(row0, CE)], r1_v, sem)
        cp_h.wait()
        cp_u.wait()
        cp_r.wait()

        def edge(e, carry2):
            r1 = [r1_v[e, pl.ds(v * 16, 16)] for v in range(HV)]
            acc_s = [jnp.zeros((16,), jnp.float32) for _ in range(HV)]
            acc_g = [jnp.zeros((16,), jnp.float32) for _ in range(HV)]
            for k in range(K):
                row = e * K + k
                for v in range(HV):
                    hv = hrows_v[row, pl.ds(v * 16, 16)]
                    uv = urows_v[row, pl.ds(v * 16, 16)]
                    r = _sigmoid16(r1[v] + uv)
                    acc_s[v] = acc_s[v] + hv
                    acc_g[v] = acc_g[v] + r * hv
            for v in range(HV):
                os_v[e, pl.ds(v * 16, 16)] = acc_s[v]
                og_v[e, pl.ds(v * 16, 16)] = acc_g[v]
            return carry2

        lax.fori_loop(0, CE, edge, 0)
        pltpu.sync_copy(os_v, sh_hbm.at[pl.ds(row0, CE)])
        pltpu.sync_copy(og_v, sg_hbm.at[pl.ds(row0, CE)])
        return carry

    lax.fori_loop(0, NCH, chunk, 0)


@functools.partial(
    pl.kernel,
    out_type=(jax.ShapeDtypeStruct((B, H), jnp.float32),
              jax.ShapeDtypeStruct((B, H), jnp.float32)),
    mesh=_mesh,
    compiler_params=_sc_params,
    scratch_types=[
        pltpu.VMEM((N * K,), jnp.int32),
        pltpu.VMEM((N,), jnp.int32),
        pltpu.VMEM((16,), jnp.int32),
        pltpu.VMEM((16,), jnp.int32),
        pltpu.VMEM((CI,), jnp.int32),
        pltpu.VMEM((16, H), jnp.float32),
        pltpu.VMEM((CI, H), jnp.float32),
        pltpu.VMEM((16, H), jnp.float32),
        pltpu.SemaphoreType.DMA,
    ],
)
def _sc_final(ng_hbm, fnode_hbm, root_hbm, emb_hbm, h_hbm, fe_hbm, mn_hbm,
              ng_v, fnode_v, root_v, idx_v, idx2_v, fe_v, hrows_v, mn_v, sem):
    w = _wid()

    @pl.when(w < B // 16)
    def _():
        base = w * 16
        pltpu.sync_copy(ng_hbm, ng_v)
        pltpu.sync_copy(fnode_hbm, fnode_v)
        pltpu.sync_copy(root_hbm.at[pl.ds(base, 16)], root_v)
        rv = root_v[...]
        idx_v[...] = plsc.load_gather(fnode_v, [rv])
        pltpu.async_copy(emb_hbm.at[idx_v], fe_v, sem).wait()
        pltpu.sync_copy(fe_v, fe_hbm.at[pl.ds(base, 16)])
        lanes = lax.iota(jnp.int32, 16)
        for k in range(K):
            ngk = plsc.load_gather(ng_v, [rv * K + k])
            plsc.store_scatter(idx2_v, [lanes * K + k], ngk)
        pltpu.async_copy(h_hbm.at[idx2_v], hrows_v, sem).wait()

        def root_i(i, carry):
            for v in range(HV):
                acc = jnp.zeros((16,), jnp.float32)
                for k in range(K):
                    acc = acc + hrows_v[i * K + k, pl.ds(v * 16, 16)]
                mn_v[i, pl.ds(v * 16, 16)] = acc
            return carry

        lax.fori_loop(0, 16, root_i, 0)
        pltpu.sync_copy(mn_v, mn_hbm.at[pl.ds(base, 16)])


TBLK = 2048
TG = EP // TBLK


def _tc_pre_body(x_ref, wcat_ref, bcat_ref, ur_ref,
                 xz_ref, r1_ref, xh_ref, h_ref, hu_ref):
    pre = jnp.dot(x_ref[...], wcat_ref[...],
                  preferred_element_type=jnp.float32) + bcat_ref[...]
    xz = pre[:, :H]
    xh = pre[:, 2 * H:]
    h = jax.nn.sigmoid(xz) * jnp.tanh(xh)
    rows = lax.broadcasted_iota(jnp.int32, h.shape, 0) + pl.program_id(0) * TBLK
    h = jnp.where(rows == 0, 0.0, h)
    xz_ref[...] = xz
    r1_ref[...] = pre[:, H:2 * H]
    xh_ref[...] = xh
    h_ref[...] = h
    hu_ref[...] = jnp.dot(h, ur_ref[...], preferred_element_type=jnp.float32)


_tc_pre = pl.pallas_call(
    _tc_pre_body,
    grid=(TG,),
    in_specs=[
        pl.BlockSpec((TBLK, H), lambda i: (i, 0)),
        pl.BlockSpec((H, 3 * H), lambda i: (0, 0)),
        pl.BlockSpec((1, 3 * H), lambda i: (0, 0)),
        pl.BlockSpec((H, H), lambda i: (0, 0)),
    ],
    out_specs=[pl.BlockSpec((TBLK, H), lambda i: (i, 0))] * 5,
    out_shape=[jax.ShapeDtypeStruct((EP, H), jnp.float32)] * 5,
)


def _tc_iter_body(sh_ref, sg_ref, xz_ref, xh_ref, wz2_ref, wh2_ref, ur_ref,
                  h_ref, hu_ref):
    sh = sh_ref[...]
    z = jax.nn.sigmoid(xz_ref[...] + jnp.dot(sh, wz2_ref[...],
                                             preferred_element_type=jnp.float32))
    p = jnp.tanh(xh_ref[...] + jnp.dot(sg_ref[...], wh2_ref[...],
                                       preferred_element_type=jnp.float32))
    h = (1.0 - z) * sh + z * p
    rows = lax.broadcasted_iota(jnp.int32, h.shape, 0) + pl.program_id(0) * TBLK
    h = jnp.where(rows == 0, 0.0, h)
    h_ref[...] = h
    hu_ref[...] = jnp.dot(h, ur_ref[...], preferred_element_type=jnp.float32)


_tc_iter = pl.pallas_call(
    _tc_iter_body,
    grid=(TG,),
    in_specs=[
        pl.BlockSpec((TBLK, H), lambda i: (i, 0)),
        pl.BlockSpec((TBLK, H), lambda i: (i, 0)),
        pl.BlockSpec((TBLK, H), lambda i: (i, 0)),
        pl.BlockSpec((TBLK, H), lambda i: (i, 0)),
        pl.BlockSpec((H, H), lambda i: (0, 0)),
        pl.BlockSpec((H, H), lambda i: (0, 0)),
        pl.BlockSpec((H, H), lambda i: (0, 0)),
    ],
    out_specs=[pl.BlockSpec((TBLK, H), lambda i: (i, 0))] * 2,
    out_shape=[jax.ShapeDtypeStruct((EP, H), jnp.float32)] * 2,
)


def _tc_final_body(fe_ref, mn_ref, w1_ref, w2_ref, b_ref, out_ref):
    acc = jnp.dot(fe_ref[...], w1_ref[...], preferred_element_type=jnp.float32)
    acc = acc + jnp.dot(mn_ref[...], w2_ref[...],
                        preferred_element_type=jnp.float32)
    out_ref[...] = jnp.maximum(acc + b_ref[...], 0.0)


_tc_final = pl.pallas_call(
    _tc_final_body,
    out_shape=jax.ShapeDtypeStruct((B, H), jnp.float32),
)


def kernel(fnode, fmess, node_graph, mess_graph, scope, emb,
           Wz_w, Wz_b, Wr_w, Ur_w, Ur_b, Wh_w, Wh_b, out_w, out_b):
    i32 = jnp.int32
    fnode = fnode.astype(i32)
    fmess_p = jnp.pad(fmess.astype(i32), (0, EP - E))
    mgf = jnp.pad(mess_graph.astype(i32).reshape(-1), (0, (EP - E) * K))
    ngf = node_graph.astype(i32).reshape(-1)
    root = scope[:, 0].astype(i32)

    wzT = Wz_w.T
    whT = Wh_w.T
    wcat = jnp.concatenate([wzT[:H], Wr_w.T, whT[:H]], axis=1)
    bcat = jnp.concatenate([Wz_b, Ur_b, Wh_b]).reshape(1, 3 * H)
    urT = Ur_w.T

    x = _sc_embed(fnode, fmess_p, emb)
    xz, r1b, xh, h, hu = _tc_pre(x, wcat, bcat, urT)
    for _ in range(DEPTH - 1):
        sh, sg = _sc_gather(mgf, h, hu, r1b)
        h, hu = _tc_iter(sh, sg, xz, xh, wzT[H:], whT[H:], urT)
    fe, mn = _sc_final(ngf, fnode, root, emb, h)
    tree = _tc_final(fe, mn, out_w.T[:H], out_w.T[H:], out_b.reshape(1, H))
    return tree, h[:E]


# speedup vs baseline: 1.8718x; 1.8112x over previous
import functools

import jax
import jax.numpy as jnp
from jax import lax
from jax.experimental import pallas as pl
from jax.experimental.pallas import tpu as pltpu
from jax.experimental.pallas import tpu_sc as plsc

N = 10000
E = 20000
K = 8
H = 128
H2 = 2 * H
DEPTH = 10
B = 256

NC, NS = 2, 16
NW = NC * NS
EP = 20480
EPW = EP // NW
CE = 16
CI = CE * K
NCH = EPW // CE
HV = H // 16

_mesh = plsc.VectorSubcoreMesh(
    core_axis_name="c", subcore_axis_name="s", num_cores=NC, num_subcores=NS)
_sc_params = pltpu.CompilerParams(needs_layout_passes=False)


def _wid():
    return lax.axis_index("s") * NC + lax.axis_index("c")


def _sigmoid16(t):
    return 1.0 / (1.0 + jnp.exp(-t))


@functools.partial(
    pl.kernel,
    out_type=jax.ShapeDtypeStruct((EP, H), jnp.float32),
    mesh=_mesh,
    compiler_params=_sc_params,
    scratch_types=[
        pltpu.VMEM((N,), jnp.int32),
        pltpu.VMEM((EPW,), jnp.int32),
        pltpu.VMEM((CE,), jnp.int32),
        pltpu.VMEM((CE, H), jnp.float32),
        pltpu.SemaphoreType.DMA,
    ],
)
def _sc_embed(fnode_hbm, fmess_hbm, emb_hbm, x_hbm,
              fnode_v, fmess_v, idx_v, rows_v, sem):
    base = _wid() * EPW
    pltpu.sync_copy(fnode_hbm, fnode_v)
    pltpu.sync_copy(fmess_hbm.at[pl.ds(base, EPW)], fmess_v)

    def chunk(c, carry):
        fm = fmess_v[pl.ds(c * CE, CE)]
        idx_v[...] = plsc.load_gather(fnode_v, [fm])
        pltpu.async_copy(emb_hbm.at[idx_v], rows_v, sem).wait()
        pltpu.sync_copy(rows_v, x_hbm.at[pl.ds(base + c * CE, CE)])
        return carry

    lax.fori_loop(0, NCH, chunk, 0)


@functools.partial(
    pl.kernel,
    out_type=jax.ShapeDtypeStruct((EP, H2), jnp.float32),
    mesh=_mesh,
    compiler_params=_sc_params,
    scratch_types=[
        pltpu.VMEM((EPW * K,), jnp.int32),
        pltpu.VMEM((CI, H2), jnp.float32),
        pltpu.VMEM((CI, H2), jnp.float32),
        pltpu.VMEM((CE, H), jnp.float32),
        pltpu.VMEM((CE, H), jnp.float32),
        pltpu.VMEM((CE, H2), jnp.float32),
        pltpu.VMEM((CE, H2), jnp.float32),
        pltpu.SemaphoreType.DMA,
        pltpu.SemaphoreType.DMA,
        pltpu.SemaphoreType.DMA,
        pltpu.SemaphoreType.DMA,
    ],
)
def _sc_gather(mg_hbm, hcat_hbm, r1_hbm, out_hbm,
               idx_v, rows_a, rows_b, r1_a, r1_b, out_a, out_b,
               gsem_a, gsem_b, osem_a, osem_b):
    base = _wid() * EPW
    pltpu.sync_copy(mg_hbm.at[pl.ds(base * K, EPW * K)], idx_v)

    def fire(c, rows_v, r1_v, sem):
        pltpu.async_copy(hcat_hbm.at[idx_v.at[pl.ds(c * CI, CI)]], rows_v, sem)
        pltpu.async_copy(r1_hbm.at[pl.ds(base + c * CE, CE)], r1_v, sem)

    def wait_g(rows_v, r1_v, sem):
        pltpu.make_async_copy(hcat_hbm.at[idx_v.at[pl.ds(0, CI)]], rows_v, sem).wait()
        pltpu.make_async_copy(r1_hbm.at[pl.ds(base, CE)], r1_v, sem).wait()

    def wait_o(out_v, sem):
        pltpu.make_async_copy(out_v, out_hbm.at[pl.ds(base, CE)], sem).wait()

    def compute(rows_v, r1_v, out_v):
        def edge(e, carry):
            r1 = [r1_v[e, pl.ds(v * 16, 16)] for v in range(HV)]
            acc_s = [jnp.zeros((16,), jnp.float32) for _ in range(HV)]
            acc_g = [jnp.zeros((16,), jnp.float32) for _ in range(HV)]
            for k in range(K):
                row = e * K + k
                for v in range(HV):
                    hv = rows_v[row, pl.ds(v * 16, 16)]
                    uv = rows_v[row, pl.ds(H + v * 16, 16)]
                    r = _sigmoid16(r1[v] + uv)
                    acc_s[v] = acc_s[v] + hv
                    acc_g[v] = acc_g[v] + r * hv
            for v in range(HV):
                out_v[e, pl.ds(v * 16, 16)] = acc_s[v]
                out_v[e, pl.ds(H + v * 16, 16)] = acc_g[v]
            return carry

        lax.fori_loop(0, CE, edge, 0)

    fire(0, rows_a, r1_a, gsem_a)

    def body(j, carry):
        c0 = 2 * j
        c1 = c0 + 1
        fire(c1, rows_b, r1_b, gsem_b)
        wait_g(rows_a, r1_a, gsem_a)
        compute(rows_a, r1_a, out_a)

        @pl.when(j > 0)
        def _():
            wait_o(out_a, osem_a)

        pltpu.async_copy(out_a, out_hbm.at[pl.ds(base + c0 * CE, CE)], osem_a)

        @pl.when(j < NCH // 2 - 1)
        def _():
            fire(c0 + 2, rows_a, r1_a, gsem_a)

        wait_g(rows_b, r1_b, gsem_b)
        compute(rows_b, r1_b, out_b)

        @pl.when(j > 0)
        def _():
            wait_o(out_b, osem_b)

        pltpu.async_copy(out_b, out_hbm.at[pl.ds(base + c1 * CE, CE)], osem_b)
        return carry

    lax.fori_loop(0, NCH // 2, body, 0)
    wait_o(out_a, osem_a)
    wait_o(out_b, osem_b)


@functools.partial(
    pl.kernel,
    out_type=(jax.ShapeDtypeStruct((B, H), jnp.float32),
              jax.ShapeDtypeStruct((B, H), jnp.float32)),
    mesh=_mesh,
    compiler_params=_sc_params,
    scratch_types=[
        pltpu.VMEM((N * K,), jnp.int32),
        pltpu.VMEM((N,), jnp.int32),
        pltpu.VMEM((16,), jnp.int32),
        pltpu.VMEM((16,), jnp.int32),
        pltpu.VMEM((CI,), jnp.int32),
        pltpu.VMEM((16, H), jnp.float32),
        pltpu.VMEM((CI, H2), jnp.float32),
        pltpu.VMEM((16, H), jnp.float32),
        pltpu.SemaphoreType.DMA,
    ],
)
def _sc_final(ng_hbm, fnode_hbm, root_hbm, emb_hbm, hcat_hbm, fe_hbm, mn_hbm,
              ng_v, fnode_v, root_v, idx_v, idx2_v, fe_v, hrows_v, mn_v, sem):
    w = _wid()

    @pl.when(w < B // 16)
    def _():
        base = w * 16
        pltpu.sync_copy(ng_hbm, ng_v)
        pltpu.sync_copy(fnode_hbm, fnode_v)
        pltpu.sync_copy(root_hbm.at[pl.ds(base, 16)], root_v)
        rv = root_v[...]
        idx_v[...] = plsc.load_gather(fnode_v, [rv])
        pltpu.async_copy(emb_hbm.at[idx_v], fe_v, sem).wait()
        pltpu.sync_copy(fe_v, fe_hbm.at[pl.ds(base, 16)])
        lanes = lax.iota(jnp.int32, 16)
        for k in range(K):
            ngk = plsc.load_gather(ng_v, [rv * K + k])
            plsc.store_scatter(idx2_v, [lanes * K + k], ngk)
        pltpu.async_copy(hcat_hbm.at[idx2_v], hrows_v, sem).wait()

        def root_i(i, carry):
            for v in range(HV):
                acc = jnp.zeros((16,), jnp.float32)
                for k in range(K):
                    acc = acc + hrows_v[i * K + k, pl.ds(v * 16, 16)]
                mn_v[i, pl.ds(v * 16, 16)] = acc
            return carry

        lax.fori_loop(0, 16, root_i, 0)
        pltpu.sync_copy(mn_v, mn_hbm.at[pl.ds(base, 16)])


TBLK = 2048
TG = EP // TBLK


def _tc_pre_body(x_ref, wcat_ref, bcat_ref, ur_ref,
                 xz_ref, r1_ref, xh_ref, hcat_ref):
    pre = jnp.dot(x_ref[...], wcat_ref[...],
                  preferred_element_type=jnp.float32) + bcat_ref[...]
    xz = pre[:, :H]
    xh = pre[:, 2 * H:]
    h = jax.nn.sigmoid(xz) * jnp.tanh(xh)
    rows = lax.broadcasted_iota(jnp.int32, h.shape, 0) + pl.program_id(0) * TBLK
    h = jnp.where(rows == 0, 0.0, h)
    xz_ref[...] = xz
    r1_ref[...] = pre[:, H:2 * H]
    xh_ref[...] = xh
    hu = jnp.dot(h, ur_ref[...], preferred_element_type=jnp.float32)
    hcat_ref[...] = jnp.concatenate([h, hu], axis=1)


_tc_pre = pl.pallas_call(
    _tc_pre_body,
    grid=(TG,),
    in_specs=[
        pl.BlockSpec((TBLK, H), lambda i: (i, 0)),
        pl.BlockSpec((H, 3 * H), lambda i: (0, 0)),
        pl.BlockSpec((1, 3 * H), lambda i: (0, 0)),
        pl.BlockSpec((H, H), lambda i: (0, 0)),
    ],
    out_specs=[
        pl.BlockSpec((TBLK, H), lambda i: (i, 0)),
        pl.BlockSpec((TBLK, H), lambda i: (i, 0)),
        pl.BlockSpec((TBLK, H), lambda i: (i, 0)),
        pl.BlockSpec((TBLK, H2), lambda i: (i, 0)),
    ],
    out_shape=[
        jax.ShapeDtypeStruct((EP, H), jnp.float32),
        jax.ShapeDtypeStruct((EP, H), jnp.float32),
        jax.ShapeDtypeStruct((EP, H), jnp.float32),
        jax.ShapeDtypeStruct((EP, H2), jnp.float32),
    ],
)


def _tc_iter_body(s_ref, xz_ref, xh_ref, wz2_ref, wh2_ref, ur_ref, hcat_ref):
    sh = s_ref[:, :H]
    z = jax.nn.sigmoid(xz_ref[...] + jnp.dot(sh, wz2_ref[...],
                                             preferred_element_type=jnp.float32))
    p = jnp.tanh(xh_ref[...] + jnp.dot(s_ref[:, H:], wh2_ref[...],
                                       preferred_element_type=jnp.float32))
    h = (1.0 - z) * sh + z * p
    rows = lax.broadcasted_iota(jnp.int32, h.shape, 0) + pl.program_id(0) * TBLK
    h = jnp.where(rows == 0, 0.0, h)
    hu = jnp.dot(h, ur_ref[...], preferred_element_type=jnp.float32)
    hcat_ref[...] = jnp.concatenate([h, hu], axis=1)


_tc_iter = pl.pallas_call(
    _tc_iter_body,
    grid=(TG,),
    in_specs=[
        pl.BlockSpec((TBLK, H2), lambda i: (i, 0)),
        pl.BlockSpec((TBLK, H), lambda i: (i, 0)),
        pl.BlockSpec((TBLK, H), lambda i: (i, 0)),
        pl.BlockSpec((H, H), lambda i: (0, 0)),
        pl.BlockSpec((H, H), lambda i: (0, 0)),
        pl.BlockSpec((H, H), lambda i: (0, 0)),
    ],
    out_specs=pl.BlockSpec((TBLK, H2), lambda i: (i, 0)),
    out_shape=jax.ShapeDtypeStruct((EP, H2), jnp.float32),
)


def _tc_final_body(fe_ref, mn_ref, w1_ref, w2_ref, b_ref, out_ref):
    acc = jnp.dot(fe_ref[...], w1_ref[...], preferred_element_type=jnp.float32)
    acc = acc + jnp.dot(mn_ref[...], w2_ref[...],
                        preferred_element_type=jnp.float32)
    out_ref[...] = jnp.maximum(acc + b_ref[...], 0.0)


_tc_final = pl.pallas_call(
    _tc_final_body,
    out_shape=jax.ShapeDtypeStruct((B, H), jnp.float32),
)


def kernel(fnode, fmess, node_graph, mess_graph, scope, emb,
           Wz_w, Wz_b, Wr_w, Ur_w, Ur_b, Wh_w, Wh_b, out_w, out_b):
    i32 = jnp.int32
    fnode = fnode.astype(i32)
    fmess_p = jnp.pad(fmess.astype(i32), (0, EP - E))
    mgf = jnp.pad(mess_graph.astype(i32).reshape(-1), (0, (EP - E) * K))
    ngf = node_graph.astype(i32).reshape(-1)
    root = scope[:, 0].astype(i32)

    wzT = Wz_w.T
    whT = Wh_w.T
    wcat = jnp.concatenate([wzT[:H], Wr_w.T, whT[:H]], axis=1)
    bcat = jnp.concatenate([Wz_b, Ur_b, Wh_b]).reshape(1, 3 * H)
    urT = Ur_w.T

    x = _sc_embed(fnode, fmess_p, emb)
    xz, r1b, xh, hcat = _tc_pre(x, wcat, bcat, urT)
    for _ in range(DEPTH - 1):
        s = _sc_gather(mgf, hcat, r1b)
        hcat = _tc_iter(s, xz, xh, wzT[H:], whT[H:], urT)
    fe, mn = _sc_final(ngf, fnode, root, emb, hcat)
    tree = _tc_final(fe, mn, out_w.T[:H], out_w.T[H:], out_b.reshape(1, H))
    return tree, hcat[:E, :H]
